# baseline (speedup 1.0000x reference)
import jax
import jax.numpy as jnp
from jax import lax
from jax.experimental import pallas as pl
from jax.experimental.pallas import tpu as pltpu
from jax.experimental.pallas import tpu_sc as plsc
import functools

_INDEX = 1024


def _make_sc_select(b, d, dtype):
    mesh = plsc.ScalarSubcoreMesh(axis_name="c", num_cores=1)

    @functools.partial(
        pl.kernel,
        mesh=mesh,
        out_type=jax.ShapeDtypeStruct((b, d), dtype),
    )
    def sc_select(x_hbm, out_hbm):
        for bi in range(b):
            pltpu.sync_copy(x_hbm.at[bi, _INDEX], out_hbm.at[bi])

    return sc_select


def kernel(x):
    b, s, d = x.shape
    return _make_sc_select(b, d, x.dtype)(x)


# speedup vs baseline: 9.6230x; 9.6230x over previous
import jax
import jax.numpy as jnp
from jax.experimental import pallas as pl
from jax.experimental.pallas import tpu as pltpu

_INDEX = 1024


def _copy_kernel(x_hbm, o_hbm, sem):
    b = o_hbm.shape[0]
    copies = [
        pltpu.make_async_copy(x_hbm.at[bi, _INDEX], o_hbm.at[bi], sem)
        for bi in range(b)
    ]
    for c in copies:
        c.start()
    for c in copies:
        c.wait()


def kernel(x):
    b, s, d = x.shape
    return pl.pallas_call(
        _copy_kernel,
        in_specs=[pl.BlockSpec(memory_space=pl.ANY)],
        out_specs=pl.BlockSpec(memory_space=pl.ANY),
        out_shape=jax.ShapeDtypeStruct((b, d), x.dtype),
        scratch_shapes=[pltpu.SemaphoreType.DMA],
    )(x)


# speedup vs baseline: 9.6361x; 1.0014x over previous
import jax
import jax.numpy as jnp
from jax.experimental import pallas as pl
from jax.experimental.pallas import tpu as pltpu

_INDEX = 1024


def _copy_kernel(x_hbm, o_hbm, sem):
    copy = pltpu.make_async_copy(x_hbm.at[:, _INDEX], o_hbm, sem)
    copy.start()
    copy.wait()


def kernel(x):
    b, s, d = x.shape
    return pl.pallas_call(
        _copy_kernel,
        in_specs=[pl.BlockSpec(memory_space=pl.ANY)],
        out_specs=pl.BlockSpec(memory_space=pl.ANY),
        out_shape=jax.ShapeDtypeStruct((b, d), x.dtype),
        scratch_shapes=[pltpu.SemaphoreType.DMA],
    )(x)
